# baseline (speedup 1.0000x reference)
import functools

import jax
import jax.numpy as jnp
from jax import lax
from jax.experimental import pallas as pl
from jax.experimental.pallas import tpu as pltpu

B, H, W, D = 16, 24, 24, 256
K = 1024
HW = H * W
COMMIT = 0.25


def _vq_body(z_ref, emb_ref, q_ref, idx_ref, loss_ref, en_ref):
    b = pl.program_id(0)
    z = z_ref[0]
    emb = emb_ref[...]

    @pl.when(b == 0)
    def _():
        en_ref[...] = jnp.sum(emb * emb, axis=1)[None, :]
        loss_ref[0, 0] = 0.0

    zn = jnp.sum(z * z, axis=1, keepdims=True)
    s2 = lax.dot_general(z * (-2.0), emb, (((1,), (1,)), ((), ())),
                         preferred_element_type=jnp.float32)
    dist = jnp.sqrt(jnp.maximum((zn + s2) + en_ref[...], 0.0))

    m = jnp.min(dist, axis=1, keepdims=True)
    kiota = lax.broadcasted_iota(jnp.int32, (HW, K), 1)
    idx = jnp.min(jnp.where(dist == m, kiota, K), axis=1)

    onehot = (kiota == idx[:, None]).astype(jnp.float32)
    qt = lax.dot_general(emb, onehot, (((0,), (1,)), ((), ())),
                         preferred_element_type=jnp.float32)
    q_ref[0] = qt.reshape(D, H, W)
    idx_ref[0] = idx.reshape(1, H, W)

    loss_ref[0, 0] += jnp.sum(m * m)

    @pl.when(b == B - 1)
    def _():
        loss_ref[0, 0] = (loss_ref[0, 0] / jnp.float32(B * HW * D)
                          * jnp.float32(COMMIT))


@functools.partial(jax.jit, static_argnames=("interpret",))
def _vq(z, embeddings, interpret=False):
    z3 = z.reshape(B, HW, D)
    q, idx, loss_sum = pl.pallas_call(
        _vq_body,
        grid=(B,),
        in_specs=[
            pl.BlockSpec((1, HW, D), lambda b: (b, 0, 0)),
            pl.BlockSpec((K, D), lambda b: (0, 0)),
        ],
        out_specs=[
            pl.BlockSpec((1, D, H, W), lambda b: (b, 0, 0, 0)),
            pl.BlockSpec((1, 1, H, W), lambda b: (b, 0, 0, 0)),
            pl.BlockSpec((1, 1), lambda b: (0, 0),
                         memory_space=pltpu.SMEM),
        ],
        out_shape=[
            jax.ShapeDtypeStruct((B, D, H, W), jnp.float32),
            jax.ShapeDtypeStruct((B, 1, H, W), jnp.int32),
            jax.ShapeDtypeStruct((1, 1), jnp.float32),
        ],
        scratch_shapes=[pltpu.VMEM((1, K), jnp.float32)],
        interpret=interpret,
    )(z3, embeddings)
    return q, idx, loss_sum.reshape(())


def kernel(z, embeddings):
    return _vq(z, embeddings)


# speedup vs baseline: 1.6716x; 1.6716x over previous
import functools

import jax
import jax.numpy as jnp
from jax import lax
from jax.experimental import pallas as pl
from jax.experimental.pallas import tpu as pltpu

B, H, W, D = 16, 24, 24, 256
K = 1024
HW = H * W
COMMIT = 0.25
GROUP = 2
ROWS = GROUP * HW
STEPS = B // GROUP


def _vq_body(z_ref, emb_ref, q_ref, idx_ref, loss_ref, en_ref):
    g = pl.program_id(0)
    z = z_ref[0]
    emb = emb_ref[...]

    @pl.when(g == 0)
    def _():
        en_ref[...] = jnp.sum(emb * emb, axis=1)[None, :]
        loss_ref[0, 0] = 0.0

    zn = jnp.sum(z * z, axis=1, keepdims=True)
    s2 = lax.dot_general(z * (-2.0), emb, (((1,), (1,)), ((), ())),
                         preferred_element_type=jnp.float32)
    dist = jnp.sqrt(jnp.maximum((zn + s2) + en_ref[...], 0.0))

    m = jnp.min(dist, axis=1, keepdims=True)
    kiota = lax.broadcasted_iota(jnp.int32, (ROWS, K), 1)
    idx = jnp.min(jnp.where(dist == m, kiota, K), axis=1)

    onehot = (kiota == idx[:, None]).astype(jnp.float32)
    for j in range(GROUP):
        oh = onehot[j * HW:(j + 1) * HW]
        q_ref[j] = lax.dot_general(emb, oh, (((0,), (1,)), ((), ())),
                                   preferred_element_type=jnp.float32)
        idx_ref[j, 0, :] = idx[j * HW:(j + 1) * HW]

    loss_ref[0, 0] += jnp.sum(m * m)

    @pl.when(g == STEPS - 1)
    def _():
        loss_ref[0, 0] = (loss_ref[0, 0] / jnp.float32(B * HW * D)
                          * jnp.float32(COMMIT))


@functools.partial(jax.jit, static_argnames=("interpret",))
def _vq(z, embeddings, interpret=False):
    z3 = z.reshape(STEPS, ROWS, D)
    q, idx, loss_sum = pl.pallas_call(
        _vq_body,
        grid=(STEPS,),
        in_specs=[
            pl.BlockSpec((1, ROWS, D), lambda g: (g, 0, 0)),
            pl.BlockSpec((K, D), lambda g: (0, 0)),
        ],
        out_specs=[
            pl.BlockSpec((GROUP, D, HW), lambda g: (g, 0, 0)),
            pl.BlockSpec((GROUP, 1, HW), lambda g: (g, 0, 0)),
            pl.BlockSpec((1, 1), lambda g: (0, 0),
                         memory_space=pltpu.SMEM),
        ],
        out_shape=[
            jax.ShapeDtypeStruct((B, D, HW), jnp.float32),
            jax.ShapeDtypeStruct((B, 1, HW), jnp.int32),
            jax.ShapeDtypeStruct((1, 1), jnp.float32),
        ],
        scratch_shapes=[pltpu.VMEM((1, K), jnp.float32)],
        interpret=interpret,
    )(z3, embeddings)
    quantized_out = q.reshape(B, D, H, W)
    indices = idx.reshape(B, 1, H, W)
    return quantized_out, indices, loss_sum.reshape(())


def kernel(z, embeddings):
    return _vq(z, embeddings)


# speedup vs baseline: 1.7772x; 1.0632x over previous
import functools

import jax
import jax.numpy as jnp
from jax import lax
from jax.experimental import pallas as pl
from jax.experimental.pallas import tpu as pltpu

B, H, W, D = 16, 24, 24, 256
K = 1024
HW = H * W
COMMIT = 0.25
GROUP = 4
ROWS = GROUP * HW
STEPS = B // GROUP


def _vq_body(z_ref, emb_ref, q_ref, idx_ref, loss_ref, en_ref):
    g = pl.program_id(0)
    z = z_ref[0]
    emb = emb_ref[...]

    @pl.when(g == 0)
    def _():
        en_ref[...] = jnp.sum(emb * emb, axis=1)[None, :]
        loss_ref[0, 0] = 0.0

    zn = jnp.sum(z * z, axis=1, keepdims=True)
    s2 = lax.dot_general(z * (-2.0), emb, (((1,), (1,)), ((), ())),
                         preferred_element_type=jnp.float32)
    dist = jnp.sqrt(jnp.maximum((zn + s2) + en_ref[...], 0.0))

    m = jnp.min(dist, axis=1, keepdims=True)
    kiota = lax.broadcasted_iota(jnp.int32, (ROWS, K), 1)
    idx = jnp.min(jnp.where(dist == m, kiota, K), axis=1)

    onehot = (kiota == idx[:, None]).astype(jnp.float32)
    for j in range(GROUP):
        oh = onehot[j * HW:(j + 1) * HW]
        q_ref[j] = lax.dot_general(emb, oh, (((0,), (1,)), ((), ())),
                                   preferred_element_type=jnp.float32)
        idx_ref[j, 0, :] = idx[j * HW:(j + 1) * HW]

    loss_ref[0, 0] += jnp.sum(m * m)

    @pl.when(g == STEPS - 1)
    def _():
        loss_ref[0, 0] = (loss_ref[0, 0] / jnp.float32(B * HW * D)
                          * jnp.float32(COMMIT))


@functools.partial(jax.jit, static_argnames=("interpret",))
def _vq(z, embeddings, interpret=False):
    z3 = z.reshape(STEPS, ROWS, D)
    q, idx, loss_sum = pl.pallas_call(
        _vq_body,
        grid=(STEPS,),
        in_specs=[
            pl.BlockSpec((1, ROWS, D), lambda g: (g, 0, 0)),
            pl.BlockSpec((K, D), lambda g: (0, 0)),
        ],
        out_specs=[
            pl.BlockSpec((GROUP, D, HW), lambda g: (g, 0, 0)),
            pl.BlockSpec((GROUP, 1, HW), lambda g: (g, 0, 0)),
            pl.BlockSpec((1, 1), lambda g: (0, 0),
                         memory_space=pltpu.SMEM),
        ],
        out_shape=[
            jax.ShapeDtypeStruct((B, D, HW), jnp.float32),
            jax.ShapeDtypeStruct((B, 1, HW), jnp.int32),
            jax.ShapeDtypeStruct((1, 1), jnp.float32),
        ],
        scratch_shapes=[pltpu.VMEM((1, K), jnp.float32)],
        interpret=interpret,
    )(z3, embeddings)
    quantized_out = q.reshape(B, D, H, W)
    indices = idx.reshape(B, 1, H, W)
    return quantized_out, indices, loss_sum.reshape(())


def kernel(z, embeddings):
    return _vq(z, embeddings)


# speedup vs baseline: 1.7782x; 1.0006x over previous
import functools

import jax
import jax.numpy as jnp
from jax import lax
from jax.experimental import pallas as pl
from jax.experimental.pallas import tpu as pltpu

B, H, W, D = 16, 24, 24, 256
K = 1024
HW = H * W
COMMIT = 0.25
GROUP = 4
ROWS = GROUP * HW
STEPS = B // GROUP


def _vq_body(z_ref, emb_ref, q_ref, idx_ref, loss_ref, en_ref):
    g = pl.program_id(0)
    z = z_ref[0]
    emb = emb_ref[...]

    @pl.when(g == 0)
    def _():
        en_ref[...] = jnp.sum(emb * emb, axis=1)[None, :]
        loss_ref[0, 0] = 0.0

    zn = jnp.sum(z * z, axis=1, keepdims=True)
    s2 = lax.dot_general(z * (-2.0), emb, (((1,), (1,)), ((), ())),
                         preferred_element_type=jnp.float32)
    dist = jnp.sqrt(jnp.maximum((zn + s2) + en_ref[...], 0.0))

    m = jnp.min(dist, axis=1, keepdims=True)
    kiota = lax.broadcasted_iota(jnp.int32, (ROWS, K), 1)
    idx = jnp.min(jnp.where(dist == m, kiota, K), axis=1)

    kiota_hw = lax.broadcasted_iota(jnp.int32, (HW, K), 1)
    for j in range(GROUP):
        idx_j = idx[j * HW:(j + 1) * HW]
        oh = (kiota_hw == idx_j[:, None]).astype(jnp.float32)
        q_ref[j] = lax.dot_general(emb, oh, (((0,), (1,)), ((), ())),
                                   preferred_element_type=jnp.float32)
        idx_ref[j, 0, :] = idx_j

    loss_ref[0, 0] += jnp.sum(m * m)

    @pl.when(g == STEPS - 1)
    def _():
        loss_ref[0, 0] = (loss_ref[0, 0] / jnp.float32(B * HW * D)
                          * jnp.float32(COMMIT))


@functools.partial(jax.jit, static_argnames=("interpret",))
def _vq(z, embeddings, interpret=False):
    z3 = z.reshape(STEPS, ROWS, D)
    q, idx, loss_sum = pl.pallas_call(
        _vq_body,
        grid=(STEPS,),
        in_specs=[
            pl.BlockSpec((1, ROWS, D), lambda g: (g, 0, 0)),
            pl.BlockSpec((K, D), lambda g: (0, 0)),
        ],
        out_specs=[
            pl.BlockSpec((GROUP, D, HW), lambda g: (g, 0, 0)),
            pl.BlockSpec((GROUP, 1, HW), lambda g: (g, 0, 0)),
            pl.BlockSpec((1, 1), lambda g: (0, 0),
                         memory_space=pltpu.SMEM),
        ],
        out_shape=[
            jax.ShapeDtypeStruct((B, D, HW), jnp.float32),
            jax.ShapeDtypeStruct((B, 1, HW), jnp.int32),
            jax.ShapeDtypeStruct((1, 1), jnp.float32),
        ],
        scratch_shapes=[pltpu.VMEM((1, K), jnp.float32)],
        interpret=interpret,
    )(z3, embeddings)
    quantized_out = q.reshape(B, D, H, W)
    indices = idx.reshape(B, 1, H, W)
    return quantized_out, indices, loss_sum.reshape(())


def kernel(z, embeddings):
    return _vq(z, embeddings)
